# Initial kernel scaffold; baseline (speedup 1.0000x reference)
import jax
import jax.numpy as jnp
from jax.experimental import pallas as pl


def kernel(x, edge_index, edge_weight, W, b, gamma, beta):
    raise NotImplementedError("write your pallas kernel here")


# speedup vs baseline: 15.9771x; 15.9771x over previous
import functools

import jax
import jax.numpy as jnp
from jax import lax
from jax.experimental import pallas as pl
from jax.experimental.pallas import tpu as pltpu
from jax.experimental.pallas import tpu_sc as plsc

_N = 10000
_E = 160000
_D = 256
_H = 128
_EPS = 1e-5

_N_PAD = 10240
_E_PAD = 163840
_ROWS = 1280
_ROWS_K1 = _ROWS // 32
_ROWS_K3 = _ROWS // 16
_BN = 512
_GRID = _N_PAD // _BN

_mesh = plsc.VectorSubcoreMesh(core_axis_name="c", subcore_axis_name="s")
_sc_params = pltpu.CompilerParams(needs_layout_passes=False)


def _deg_kernel(dst2, ew2, out, dstall, ewall, degbuf, tmp, res, shared):
    c = lax.axis_index("c")
    s = lax.axis_index("s")
    w = s * 2 + c

    @functools.partial(plsc.parallel_loop, 0, _N_PAD // 16)
    def _zero(i):
        degbuf[pl.ds(i * 16, 16)] = jnp.zeros((16,), jnp.float32)

    pltpu.sync_copy(dst2.at[pl.ds(w * _ROWS_K1, _ROWS_K1)], dstall)
    pltpu.sync_copy(ew2.at[pl.ds(w * _ROWS_K1, _ROWS_K1)], ewall)

    lanes = lax.iota(jnp.int32, 16)

    def _chunk(i, carry):
        for j in range(8):
            dv = dstall[i, pl.ds(j * 16, 16)]
            ev = ewall[i, pl.ds(j * 16, 16)]
            for l in range(16):
                plsc.addupdate_scatter(degbuf, [dv], ev, mask=lanes == l)
        return carry

    lax.fori_loop(0, _ROWS_K1, _chunk, 0)

    pltpu.sync_copy(degbuf, shared.at[s])
    plsc.subcore_barrier()

    base = s * (_N_PAD // 16)
    for p in range(16):
        pltpu.sync_copy(shared.at[p, pl.ds(base, _N_PAD // 16)], tmp.at[p])

    def _red(i, carry):
        acc = tmp[0, pl.ds(i * 16, 16)]
        for p in range(1, 16):
            acc = acc + tmp[p, pl.ds(i * 16, 16)]
        res[pl.ds(i * 16, 16)] = acc
        return carry

    lax.fori_loop(0, _N_PAD // 16 // 16, _red, 0)
    pltpu.sync_copy(res, out.at[c, pl.ds(base, _N_PAD // 16)])


_deg_call = functools.partial(
    pl.kernel,
    out_type=jax.ShapeDtypeStruct((2, _N_PAD), jnp.float32),
    mesh=_mesh,
    compiler_params=_sc_params,
    scratch_types=[
        pltpu.VMEM((_ROWS_K1, 128), jnp.int32),
        pltpu.VMEM((_ROWS_K1, 128), jnp.float32),
        pltpu.VMEM((_N_PAD,), jnp.float32),
        pltpu.VMEM((16, _N_PAD // 16), jnp.float32),
        pltpu.VMEM((_N_PAD // 16,), jnp.float32),
        pltpu.VMEM_SHARED((16, _N_PAD), jnp.float32),
    ],
)(_deg_kernel)


def _mm_body(x_ref, w_ref, deg_ref, h2_ref, dis_ref):
    xb = x_ref[...]
    wm = w_ref[...]
    h = lax.dot_general(xb, wm, (((1,), (1,)), ((), ())),
                        preferred_element_type=jnp.float32)
    deg = deg_ref[0, :] + deg_ref[1, :] + 1.0
    dis = lax.rsqrt(deg)
    hs = h * dis[:, None]
    h2_ref[0] = hs[:, :_H]
    h2_ref[1] = hs[:, _H:]
    dis_ref[...] = dis[:, None]


def _mm_call(xp, W, deg2):
    return pl.pallas_call(
        _mm_body,
        grid=(_GRID,),
        in_specs=[
            pl.BlockSpec((_BN, _D), lambda i: (i, 0)),
            pl.BlockSpec((_D, _D), lambda i: (0, 0)),
            pl.BlockSpec((2, _BN), lambda i: (0, i)),
        ],
        out_specs=[
            pl.BlockSpec((2, _BN, _H), lambda i: (0, i, 0)),
            pl.BlockSpec((_BN, 1), lambda i: (i, 0)),
        ],
        out_shape=[
            jax.ShapeDtypeStruct((2, _N_PAD, _H), jnp.float32),
            jax.ShapeDtypeStruct((_N_PAD, 1), jnp.float32),
        ],
    )(xp, W, deg2)


def _agg_kernel(h2flat, src2, dst2, ew2, out, srcall, dstall, ewall,
                srcoff, rowbuf, acc, sem):
    c = lax.axis_index("c")
    s = lax.axis_index("s")

    @functools.partial(plsc.parallel_loop, 0, 128)
    def _zrow(r):
        for k in range(8):
            rowbuf[r, pl.ds(k * 16, 16)] = jnp.zeros((16,), jnp.float32)

    zbase = s * (_N_PAD // 16)
    for k in range(5):
        pltpu.sync_copy(rowbuf, acc.at[pl.ds(zbase + k * 128, 128)])
    plsc.subcore_barrier()

    ebase = s * _ROWS_K3
    pltpu.sync_copy(src2.at[pl.ds(ebase, _ROWS_K3)], srcall)
    pltpu.sync_copy(dst2.at[pl.ds(ebase, _ROWS_K3)], dstall)
    pltpu.sync_copy(ew2.at[pl.ds(ebase, _ROWS_K3)], ewall)

    off = c * _N_PAD

    def _chunk(i, carry):
        for j in range(8):
            srcoff[pl.ds(j * 16, 16)] = srcall[i, pl.ds(j * 16, 16)] + off
        pltpu.async_copy(h2flat.at[srcoff], rowbuf, sem).wait()

        @functools.partial(plsc.parallel_loop, 0, 128, unroll=8)
        def _scale(e):
            g = (e // 16) * 16
            grp = ewall[i, pl.ds(g, 16)]
            lane = jnp.zeros((16,), jnp.int32) + (e % 16)
            cf = jnp.take(grp, lane, mode="promise_in_bounds")
            for k in range(8):
                rowbuf[e, pl.ds(k * 16, 16)] = rowbuf[e, pl.ds(k * 16, 16)] * cf

        pltpu.sync_copy(rowbuf, acc.at[dstall.at[i]], add=True)
        return carry

    lax.fori_loop(0, _ROWS_K3, _chunk, 0)
    plsc.subcore_barrier()

    obase = s * (_N_PAD // 16)
    pltpu.sync_copy(acc.at[pl.ds(obase, _N_PAD // 16)],
                    out.at[c, pl.ds(obase, _N_PAD // 16)])


_agg_call = functools.partial(
    pl.kernel,
    out_type=jax.ShapeDtypeStruct((2, _N_PAD, _H), jnp.float32),
    mesh=_mesh,
    compiler_params=_sc_params,
    scratch_types=[
        pltpu.VMEM((_ROWS_K3, 128), jnp.int32),
        pltpu.VMEM((_ROWS_K3, 128), jnp.int32),
        pltpu.VMEM((_ROWS_K3, 128), jnp.float32),
        pltpu.VMEM((128,), jnp.int32),
        pltpu.VMEM((128, _H), jnp.float32),
        pltpu.VMEM_SHARED((_N_PAD, _H), jnp.float32),
        pltpu.SemaphoreType.DMA,
    ],
)(_agg_kernel)


def _fin_body(agg_ref, h2_ref, dis_ref, b_ref, g_ref, be_ref, o_ref):
    s0 = agg_ref[0] + h2_ref[0]
    s1 = agg_ref[1] + h2_ref[1]
    sfull = jnp.concatenate([s0, s1], axis=1)
    sfull = sfull * dis_ref[...] + b_ref[...]
    mu = jnp.mean(sfull, axis=-1, keepdims=True)
    zc = sfull - mu
    var = jnp.mean(zc * zc, axis=-1, keepdims=True)
    o = zc * lax.rsqrt(var + _EPS) * g_ref[...] + be_ref[...]
    o_ref[...] = jnp.maximum(o, 0.0)


def _fin_call(agg, h2, dis, b2, g2, be2):
    return pl.pallas_call(
        _fin_body,
        grid=(_GRID,),
        in_specs=[
            pl.BlockSpec((2, _BN, _H), lambda i: (0, i, 0)),
            pl.BlockSpec((2, _BN, _H), lambda i: (0, i, 0)),
            pl.BlockSpec((_BN, 1), lambda i: (i, 0)),
            pl.BlockSpec((1, _D), lambda i: (0, 0)),
            pl.BlockSpec((1, _D), lambda i: (0, 0)),
            pl.BlockSpec((1, _D), lambda i: (0, 0)),
        ],
        out_specs=pl.BlockSpec((_BN, _D), lambda i: (i, 0)),
        out_shape=jax.ShapeDtypeStruct((_N_PAD, _D), jnp.float32),
    )(agg, h2, dis, b2, g2, be2)


def kernel(x, edge_index, edge_weight, W, b, gamma, beta):
    src = edge_index[0].astype(jnp.int32)
    dst = edge_index[1].astype(jnp.int32)
    ew = edge_weight.astype(jnp.float32)

    npad = _E_PAD - _E
    fill = jnp.arange(npad, dtype=jnp.int32) % _N
    src2 = jnp.concatenate([src, fill]).reshape(_ROWS, 128)
    dst2 = jnp.concatenate([dst, fill]).reshape(_ROWS, 128)
    ew2 = jnp.concatenate([ew, jnp.zeros((npad,), jnp.float32)]).reshape(
        _ROWS, 128)

    deg2 = _deg_call(dst2, ew2)

    xp = jnp.pad(x, ((0, _N_PAD - _N), (0, 0)))
    h2, dis = _mm_call(xp, W, deg2)

    h2flat = h2.reshape(2 * _N_PAD, _H)
    agg = _agg_call(h2flat, src2, dst2, ew2)

    out = _fin_call(agg, h2, dis,
                    b.reshape(1, _D), gamma.reshape(1, _D),
                    beta.reshape(1, _D))
    return out[:_N]


# speedup vs baseline: 23.1042x; 1.4461x over previous
import functools

import jax
import jax.numpy as jnp
from jax import lax
from jax.experimental import pallas as pl
from jax.experimental.pallas import tpu as pltpu
from jax.experimental.pallas import tpu_sc as plsc

_N = 10000
_E = 160000
_D = 256
_H = 128
_EPS = 1e-5

_N_PAD = 10240
_E_PAD = 163840
_ROWS = 1280
_ROWS_K1 = _ROWS // 32
_ROWS_K3 = _ROWS // 16
_BN = 512
_GRID = _N_PAD // _BN

_mesh = plsc.VectorSubcoreMesh(core_axis_name="c", subcore_axis_name="s")
_sc_params = pltpu.CompilerParams(needs_layout_passes=False)


def _deg_kernel(dst2, ew2, out, dstall, ewall, degbuf, tmp, res, shared):
    c = lax.axis_index("c")
    s = lax.axis_index("s")
    w = s * 2 + c

    @functools.partial(plsc.parallel_loop, 0, _N_PAD // 16)
    def _zero(i):
        degbuf[pl.ds(i * 16, 16)] = jnp.zeros((16,), jnp.float32)

    pltpu.sync_copy(dst2.at[pl.ds(w * _ROWS_K1, _ROWS_K1)], dstall)
    pltpu.sync_copy(ew2.at[pl.ds(w * _ROWS_K1, _ROWS_K1)], ewall)

    lanes = lax.iota(jnp.int32, 16)

    def _chunk(i, carry):
        for j in range(8):
            dv = dstall[i, 0, pl.ds(j * 16, 16)]
            ev = ewall[i, 0, pl.ds(j * 16, 16)]
            for l in range(16):
                plsc.addupdate_scatter(degbuf, [dv], ev, mask=lanes == l)
        return carry

    lax.fori_loop(0, _ROWS_K1, _chunk, 0)

    pltpu.sync_copy(degbuf, shared.at[s])
    plsc.subcore_barrier()

    base = s * (_N_PAD // 16)
    for p in range(16):
        pltpu.sync_copy(shared.at[p, pl.ds(base, _N_PAD // 16)], tmp.at[p])

    def _red(i, carry):
        acc = tmp[0, pl.ds(i * 16, 16)]
        for p in range(1, 16):
            acc = acc + tmp[p, pl.ds(i * 16, 16)]
        res[pl.ds(i * 16, 16)] = acc
        return carry

    lax.fori_loop(0, _N_PAD // 16 // 16, _red, 0)
    pltpu.sync_copy(res, out.at[c, pl.ds(base, _N_PAD // 16)])


_deg_call = functools.partial(
    pl.kernel,
    out_type=jax.ShapeDtypeStruct((2, _N_PAD), jnp.float32),
    mesh=_mesh,
    compiler_params=_sc_params,
    scratch_types=[
        pltpu.VMEM((_ROWS_K1, 1, 128), jnp.int32),
        pltpu.VMEM((_ROWS_K1, 1, 128), jnp.float32),
        pltpu.VMEM((_N_PAD,), jnp.float32),
        pltpu.VMEM((16, _N_PAD // 16), jnp.float32),
        pltpu.VMEM((_N_PAD // 16,), jnp.float32),
        pltpu.VMEM_SHARED((16, _N_PAD), jnp.float32),
    ],
)(_deg_kernel)


def _mm_body(x_ref, w_ref, deg_ref, h2_ref, dis_ref):
    xb = x_ref[...]
    wm = w_ref[...]
    h = lax.dot_general(xb, wm, (((1,), (1,)), ((), ())),
                        preferred_element_type=jnp.float32)
    deg = deg_ref[0, :] + deg_ref[1, :] + 1.0
    dis = lax.rsqrt(deg)
    hs = h * dis[:, None]
    h2_ref[0] = hs[:, :_H]
    h2_ref[1] = hs[:, _H:]
    dis_ref[...] = dis[:, None]


def _mm_call(xp, W, deg2):
    return pl.pallas_call(
        _mm_body,
        grid=(_GRID,),
        in_specs=[
            pl.BlockSpec((_BN, _D), lambda i: (i, 0)),
            pl.BlockSpec((_D, _D), lambda i: (0, 0)),
            pl.BlockSpec((2, _BN), lambda i: (0, i)),
        ],
        out_specs=[
            pl.BlockSpec((2, _BN, _H), lambda i: (0, i, 0)),
            pl.BlockSpec((_BN, 1), lambda i: (i, 0)),
        ],
        out_shape=[
            jax.ShapeDtypeStruct((2, _N_PAD, _H), jnp.float32),
            jax.ShapeDtypeStruct((_N_PAD, 1), jnp.float32),
        ],
    )(xp, W, deg2)


_ACC_ROWS = 10000
_ZROWS = _ACC_ROWS // 16


def _agg_kernel(h2flat, src2, dst2, ew2, out,
                gbuf0, gbuf1, gbuf2,
                srcb0, srcb1, srcb2, dstb0, dstb1, dstb2,
                ewb0, ewb1, ewb2, sdst0, sdst1, soff0, soff1, acc,
                gsem0, gsem1, gsem2, ssem0, ssem1, esem0, esem1, esem2):
    c = lax.axis_index("c")
    s = lax.axis_index("s")

    gb = (gbuf0, gbuf1, gbuf2)
    srcb = (srcb0, srcb1, srcb2)
    dstb = (dstb0, dstb1, dstb2)
    ewb = (ewb0, ewb1, ewb2)
    sdst = (sdst0, sdst1)
    soff = (soff0, soff1)
    gsem = (gsem0, gsem1, gsem2)
    ssem = (ssem0, ssem1)
    esem = (esem0, esem1, esem2)

    @functools.partial(plsc.parallel_loop, 0, 128)
    def _zrow(r):
        for k in range(8):
            gbuf0[r, pl.ds(k * 16, 16)] = jnp.zeros((16,), jnp.float32)

    zbase = jnp.minimum(s * 632, _ACC_ROWS - 632)
    for k in range(4):
        pltpu.sync_copy(gbuf0, acc.at[pl.ds(zbase + k * 128, 128)])
    pltpu.sync_copy(gbuf0.at[pl.ds(0, 120)],
                    acc.at[pl.ds(zbase + 512, 120)])
    plsc.subcore_barrier()

    ebase = s * _ROWS_K3
    off = c * _N_PAD

    def _edge_dma_start(i, q):
        r = ebase + i
        pltpu.async_copy(src2.at[r], srcb[q], esem[q])
        pltpu.async_copy(dst2.at[r], dstb[q], esem[q])
        pltpu.async_copy(ew2.at[r], ewb[q], esem[q])

    def _edge_dma_wait(i, q):
        r = ebase + i
        pltpu.make_async_copy(src2.at[r], srcb[q], esem[q]).wait()
        pltpu.make_async_copy(dst2.at[r], dstb[q], esem[q]).wait()
        pltpu.make_async_copy(ew2.at[r], ewb[q], esem[q]).wait()

    def _prep_and_gather(i, q, p):
        _edge_dma_wait(i, q)
        for j in range(8):
            soff[p][pl.ds(j * 16, 16)] = srcb[q][0, pl.ds(j * 16, 16)] + off
        pltpu.async_copy(h2flat.at[soff[p]], gb[q], gsem[q])

    def _wait_scatter(q, p):
        pltpu.make_async_copy(gb[q], acc.at[sdst[p].at[0]], ssem[p]).wait()

    def _slot(i, q, has_next, has_next2, guard_first):
        b = q % 3
        p = q % 2
        q1 = (q + 1) % 3
        p1 = (q + 1) % 2
        q2 = (q + 2) % 3

        if guard_first:
            @pl.when(i >= 2)
            def _():
                _wait_scatter(q1, p)
        else:
            _wait_scatter(q1, p)

        if has_next:
            _prep_and_gather(i + 1, q1, p1)
        if has_next2:
            _edge_dma_start(i + 2, q2)

        pltpu.make_async_copy(h2flat.at[soff[p]], gb[b], gsem[b]).wait()

        @functools.partial(plsc.parallel_loop, 0, 128, unroll=8)
        def _scale(e):
            g = (e // 16) * 16
            grp = ewb[b][0, pl.ds(g, 16)]
            lane = jnp.zeros((16,), jnp.int32) + (e % 16)
            cf = jnp.take(grp, lane, mode="promise_in_bounds")
            for k in range(8):
                gb[b][e, pl.ds(k * 16, 16)] = gb[b][e, pl.ds(k * 16, 16)] * cf

        for j in range(8):
            sdst[p][0, pl.ds(j * 16, 16)] = dstb[b][0, pl.ds(j * 16, 16)]
        pltpu.async_copy(gb[b], acc.at[sdst[p].at[0]], ssem[p], add=True)

    _edge_dma_start(0, 0)
    _edge_dma_start(1, 1)
    _prep_and_gather(0, 0, 0)

    def _round(t, carry):
        i0 = t * 6
        for q in range(6):
            _slot(i0 + q, q, True, True, True)
        return carry

    lax.fori_loop(0, (_ROWS_K3 - 2) // 6, _round, 0)

    _slot(_ROWS_K3 - 2, 0, True, False, False)
    _slot(_ROWS_K3 - 1, 1, False, False, False)
    _wait_scatter(0, 0)
    _wait_scatter(1, 1)
    plsc.subcore_barrier()

    obase = jnp.minimum(s * 632, _ACC_ROWS - 632)
    pltpu.sync_copy(acc.at[pl.ds(obase, 632)],
                    out.at[c, pl.ds(obase, 632)])


_agg_call = functools.partial(
    pl.kernel,
    out_type=jax.ShapeDtypeStruct((2, _N_PAD, _H), jnp.float32),
    mesh=_mesh,
    compiler_params=_sc_params,
    scratch_types=[
        pltpu.VMEM((128, _H), jnp.float32),
        pltpu.VMEM((128, _H), jnp.float32),
        pltpu.VMEM((128, _H), jnp.float32),
        pltpu.VMEM((1, 128), jnp.int32),
        pltpu.VMEM((1, 128), jnp.int32),
        pltpu.VMEM((1, 128), jnp.int32),
        pltpu.VMEM((1, 128), jnp.int32),
        pltpu.VMEM((1, 128), jnp.int32),
        pltpu.VMEM((1, 128), jnp.int32),
        pltpu.VMEM((1, 128), jnp.float32),
        pltpu.VMEM((1, 128), jnp.float32),
        pltpu.VMEM((1, 128), jnp.float32),
        pltpu.VMEM((1, 128), jnp.int32),
        pltpu.VMEM((1, 128), jnp.int32),
        pltpu.VMEM((128,), jnp.int32),
        pltpu.VMEM((128,), jnp.int32),
        pltpu.VMEM_SHARED((_ACC_ROWS, _H), jnp.float32),
        pltpu.SemaphoreType.DMA,
        pltpu.SemaphoreType.DMA,
        pltpu.SemaphoreType.DMA,
        pltpu.SemaphoreType.DMA,
        pltpu.SemaphoreType.DMA,
        pltpu.SemaphoreType.DMA,
        pltpu.SemaphoreType.DMA,
        pltpu.SemaphoreType.DMA,
    ],
)(_agg_kernel)


def _fin_body(agg_ref, h2_ref, dis_ref, b_ref, g_ref, be_ref, o_ref):
    s0 = agg_ref[0] + h2_ref[0]
    s1 = agg_ref[1] + h2_ref[1]
    sfull = jnp.concatenate([s0, s1], axis=1)
    sfull = sfull * dis_ref[...] + b_ref[...]
    mu = jnp.mean(sfull, axis=-1, keepdims=True)
    zc = sfull - mu
    var = jnp.mean(zc * zc, axis=-1, keepdims=True)
    o = zc * lax.rsqrt(var + _EPS) * g_ref[...] + be_ref[...]
    o_ref[...] = jnp.maximum(o, 0.0)


def _fin_call(agg, h2, dis, b2, g2, be2):
    return pl.pallas_call(
        _fin_body,
        grid=(_GRID,),
        in_specs=[
            pl.BlockSpec((2, _BN, _H), lambda i: (0, i, 0)),
            pl.BlockSpec((2, _BN, _H), lambda i: (0, i, 0)),
            pl.BlockSpec((_BN, 1), lambda i: (i, 0)),
            pl.BlockSpec((1, _D), lambda i: (0, 0)),
            pl.BlockSpec((1, _D), lambda i: (0, 0)),
            pl.BlockSpec((1, _D), lambda i: (0, 0)),
        ],
        out_specs=pl.BlockSpec((_BN, _D), lambda i: (i, 0)),
        out_shape=jax.ShapeDtypeStruct((_N_PAD, _D), jnp.float32),
    )(agg, h2, dis, b2, g2, be2)


def kernel(x, edge_index, edge_weight, W, b, gamma, beta):
    src = edge_index[0].astype(jnp.int32)
    dst = edge_index[1].astype(jnp.int32)
    ew = edge_weight.astype(jnp.float32)

    npad = _E_PAD - _E
    fill = jnp.arange(npad, dtype=jnp.int32) % _N
    src2 = jnp.concatenate([src, fill]).reshape(_ROWS, 1, 128)
    dst2 = jnp.concatenate([dst, fill]).reshape(_ROWS, 1, 128)
    ew2 = jnp.concatenate([ew, jnp.zeros((npad,), jnp.float32)]).reshape(
        _ROWS, 1, 128)

    deg2 = _deg_call(dst2, ew2)

    xp = jnp.pad(x, ((0, _N_PAD - _N), (0, 0)))
    h2, dis = _mm_call(xp, W, deg2)

    h2flat = h2.reshape(2 * _N_PAD, _H)
    agg = _agg_call(h2flat, src2, dst2, ew2)

    out = _fin_call(agg, h2, dis,
                    b.reshape(1, _D), gamma.reshape(1, _D),
                    beta.reshape(1, _D))
    return out[:_N]
